# baseline (speedup 1.0000x reference)
import functools

import jax
import jax.numpy as jnp
from jax import lax
from jax.experimental import pallas as pl
from jax.experimental.pallas import tpu as pltpu
from jax.experimental.pallas import tpu_sc as plsc

NC = 2
NS = 16
NW = NC * NS

IDX_MINOR = 128


def _combine_body(sym_ref, pos_ref, t_ref):
    t_ref[...] = pos_ref[...] + sym_ref[...]


def _combine_tables(sym_padded, pos):
    return pl.pallas_call(
        _combine_body,
        out_shape=jax.ShapeDtypeStruct(pos.shape, pos.dtype),
    )(sym_padded, pos)


def _sc_gather(table, idx3, dist3, b, l, d):
    rows_per_w = b // NW
    g_per_row = l // IDX_MINOR

    mesh = plsc.VectorSubcoreMesh(
        core_axis_name="c", subcore_axis_name="s", num_cores=NC,
        num_subcores=NS)

    @functools.partial(
        pl.kernel,
        out_type=jax.ShapeDtypeStruct((b, l, d), jnp.float32),
        mesh=mesh,
        compiler_params=pltpu.CompilerParams(needs_layout_passes=False,
                                             use_tc_tiling_on_sc=False),
        scratch_types=[
            pltpu.VMEM((g_per_row, IDX_MINOR), jnp.int32),
            pltpu.VMEM((g_per_row, IDX_MINOR), jnp.float32),
            pltpu.VMEM((l, d), jnp.float32),
            pltpu.SemaphoreType.DMA,
        ],
    )
    def run(t_hbm, idx_hbm, dist_hbm, out_hbm, idx_v, dist_v, rows_v, sem):
        wid = lax.axis_index("s") * NC + lax.axis_index("c")
        base = wid * rows_per_w

        def row_body(c, _):
            bi = base + c
            pltpu.sync_copy(idx_hbm.at[bi], idx_v)
            pltpu.sync_copy(dist_hbm.at[bi], dist_v)
            copies = []
            for j in range(g_per_row):
                copies.append(pltpu.async_copy(
                    t_hbm.at[idx_v.at[j]],
                    rows_v.at[pl.ds(j * IDX_MINOR, IDX_MINOR)],
                    sem))
            for cp in copies:
                cp.wait()

            col_ids = jnp.full((16,), d - 1, jnp.int32)
            lane = lax.iota(jnp.int32, 16)

            def fix_body(g, _):
                row_ids = g * 16 + lane
                dval = dist_v[g // 8, pl.ds(pl.multiple_of((g % 8) * 16, 16),
                                            16)]
                plsc.addupdate_scatter(rows_v, [row_ids, col_ids], dval)
                return 0

            lax.fori_loop(0, l // 16, fix_body, 0)
            pltpu.sync_copy(rows_v, out_hbm.at[bi])
            return 0

        lax.fori_loop(0, rows_per_w, row_body, 0)

    return run(table, idx3, dist3)


def kernel(used_symbols, distribution, symbol_embeddings, positional_embeddings):
    b, l = used_symbols.shape
    v, dm1 = symbol_embeddings.shape
    d = dm1 + 1

    sym_padded = jnp.pad(symbol_embeddings, ((0, 0), (0, 1)))
    table = _combine_tables(sym_padded, positional_embeddings)

    idx3 = used_symbols.astype(jnp.int32).reshape(b, l // IDX_MINOR, IDX_MINOR)
    dist3 = distribution.reshape(b, l // IDX_MINOR, IDX_MINOR)
    return _sc_gather(table, idx3, dist3, b, l, d)


# speedup vs baseline: 1.0054x; 1.0054x over previous
import functools

import jax
import jax.numpy as jnp
from jax import lax
from jax.experimental import pallas as pl
from jax.experimental.pallas import tpu as pltpu
from jax.experimental.pallas import tpu_sc as plsc

NC = 2
NS = 16
NW = NC * NS

IDX_MINOR = 128


def _combine_body(sym_ref, pos_ref, t_ref):
    t_ref[...] = pos_ref[...] + sym_ref[...]


def _combine_tables(sym_padded, pos):
    return pl.pallas_call(
        _combine_body,
        out_shape=jax.ShapeDtypeStruct(pos.shape, pos.dtype),
    )(sym_padded, pos)


def _sc_gather(table, idx1, dist1, b, l, d):
    rows_per_w = b // NW
    g_per_row = l // IDX_MINOR

    mesh = plsc.VectorSubcoreMesh(
        core_axis_name="c", subcore_axis_name="s", num_cores=NC,
        num_subcores=NS)

    @functools.partial(
        pl.kernel,
        out_type=jax.ShapeDtypeStruct((b, l, d), jnp.float32),
        mesh=mesh,
        compiler_params=pltpu.CompilerParams(needs_layout_passes=False,
                                             use_tc_tiling_on_sc=False),
        scratch_types=[
            pltpu.VMEM((l,), jnp.int32),
            pltpu.VMEM((l,), jnp.float32),
            pltpu.VMEM((l, d), jnp.float32),
            pltpu.SemaphoreType.DMA,
        ],
    )
    def run(t_hbm, idx_hbm, dist_hbm, out_hbm, idx_v, dist_v, rows_v, sem):
        wid = lax.axis_index("s") * NC + lax.axis_index("c")
        base = wid * rows_per_w

        def row_body(c, _):
            bi = base + c
            off = pl.multiple_of(bi * l, l)
            pltpu.sync_copy(idx_hbm.at[pl.ds(off, l)], idx_v)
            pltpu.sync_copy(dist_hbm.at[pl.ds(off, l)], dist_v)
            copies = []
            for j in range(g_per_row):
                copies.append(pltpu.async_copy(
                    t_hbm.at[idx_v.at[pl.ds(j * IDX_MINOR, IDX_MINOR)]],
                    rows_v.at[pl.ds(j * IDX_MINOR, IDX_MINOR)],
                    sem))
            for cp in copies:
                cp.wait()

            col_ids = jnp.full((16,), d - 1, jnp.int32)
            lane = lax.iota(jnp.int32, 16)

            def fix_body(g, _):
                row_ids = g * 16 + lane
                dval = dist_v[pl.ds(pl.multiple_of(g * 16, 16), 16)]
                plsc.addupdate_scatter(rows_v, [row_ids, col_ids], dval)
                return 0

            lax.fori_loop(0, l // 16, fix_body, 0)
            pltpu.sync_copy(rows_v, out_hbm.at[bi])
            return 0

        lax.fori_loop(0, rows_per_w, row_body, 0)

    return run(table, idx1, dist1)


def kernel(used_symbols, distribution, symbol_embeddings, positional_embeddings):
    b, l = used_symbols.shape
    v, dm1 = symbol_embeddings.shape
    d = dm1 + 1

    sym_padded = jnp.pad(symbol_embeddings, ((0, 0), (0, 1)))
    table = _combine_tables(sym_padded, positional_embeddings)

    idx1 = used_symbols.astype(jnp.int32).reshape(b * l)
    dist1 = distribution.reshape(b * l)
    return _sc_gather(table, idx1, dist1, b, l, d)
